# baseline (speedup 1.0000x reference)
import jax
import jax.numpy as jnp
from jax import lax
from jax.experimental import pallas as pl
from jax.experimental.pallas import tpu as pltpu
from jax.experimental.pallas import tpu_sc as plsc

N_NODES = 10000
N_EDGES = 320000
D = 128
D_EDGE = 16
BN_EPS = 1e-5

NC = 2
NS = 16
DH = D // NC
EPT = N_EDGES // NS
C = 80
NCHUNK = EPT // C
N_PAD = 10240
ROWS_PT = N_PAD // NS
LANES = 16
NBUF = 5
NGRP = NCHUNK // NBUF


_BLK1 = 3200


def _edge_proj_body(a_ref, w_ref, b_ref, o_ref):
    o_ref[0] = (
        jnp.dot(a_ref[...], w_ref[0], preferred_element_type=jnp.float32)
        + b_ref[0]
    )


def _edge_proj(edge_attr, W_e, b_e):
    w2 = W_e.reshape(D_EDGE, NC, DH).swapaxes(0, 1)
    b2 = b_e.reshape(NC, 1, DH)
    return pl.pallas_call(
        _edge_proj_body,
        grid=(N_EDGES // _BLK1, NC),
        in_specs=[
            pl.BlockSpec((_BLK1, D_EDGE), lambda i, h: (i, 0)),
            pl.BlockSpec((1, D_EDGE, DH), lambda i, h: (h, 0, 0)),
            pl.BlockSpec((1, 1, DH), lambda i, h: (h, 0, 0)),
        ],
        out_specs=pl.BlockSpec((1, _BLK1, DH), lambda i, h: (h, i, 0)),
        out_shape=jax.ShapeDtypeStruct((NC, N_EDGES, DH), jnp.float32),
    )(edge_attr, w2, b2)


def _sc_body(x_hbm, src_hbm, dst_hbm, e_hbm, out_hbm,
             sbuf, dbuf, msgb, agg_sh, ssem, dsem, esem, gsem):
    cid = lax.axis_index("c")
    sid = lax.axis_index("s")
    wid = cid * NS + sid
    eoff = cid * N_EDGES + sid * EPT

    def idx_load(k, b):
        pltpu.async_copy(src_hbm.at[wid, k], sbuf.at[b], ssem.at[b])
        pltpu.async_copy(dst_hbm.at[wid, k], dbuf.at[b], dsem.at[b])

    def wait_s(b):
        pltpu.make_async_copy(src_hbm.at[0, 0], sbuf.at[b], ssem.at[b]).wait()

    def wait_d(b):
        pltpu.make_async_copy(dst_hbm.at[0, 0], dbuf.at[b], dsem.at[b]).wait()

    def e_load(k, b):
        base = pl.multiple_of(eoff + k * C, 8)
        pltpu.async_copy(e_hbm.at[pl.ds(base, C)], msgb.at[b], esem.at[b])

    def wait_e(b):
        pltpu.make_async_copy(e_hbm.at[pl.ds(0, C)], msgb.at[b],
                              esem.at[b]).wait()

    def gather(b):
        pltpu.async_copy(x_hbm.at[sbuf.at[b]], msgb.at[b], gsem.at[b],
                         add=True)

    def wait_g(b):
        pltpu.make_async_copy(e_hbm.at[pl.ds(0, C)], msgb.at[b],
                              gsem.at[b]).wait()

    def relu(b):
        def _row(r, c2):
            for rr in range(2):
                for j in range(DH // LANES):
                    sl = pl.ds(j * LANES, LANES)
                    msgb[b, 2 * r + rr, sl] = jnp.maximum(
                        msgb[b, 2 * r + rr, sl], 0.0)
            return c2
        lax.fori_loop(0, C // 2, _row, 0)

    zero = jnp.zeros((LANES,), jnp.float32)

    def _zrow(r, carry):
        for j in range(DH // LANES):
            msgb[0, r, pl.ds(j * LANES, LANES)] = zero
        return carry

    lax.fori_loop(0, C, _zrow, 0)
    for i in range(ROWS_PT // C):
        pltpu.sync_copy(msgb.at[0],
                        agg_sh.at[pl.ds(sid * ROWS_PT + i * C, C)])

    for b in range(NBUF):
        idx_load(b, b)
        e_load(b, b)
    plsc.subcore_barrier()

    wait_s(0)
    wait_e(0)
    gather(0)

    def _group(g, carry):
        for b in range(NBUF):
            k = g * NBUF + b
            bn = (b + 1) % NBUF
            wait_s(bn)
            wait_e(bn)
            gather(bn)
            wait_g(b)
            relu(b)
            wait_d(b)
            pltpu.sync_copy(msgb.at[b], agg_sh.at[dbuf.at[b]], add=True)
            idx_load(k + NBUF, b)
            e_load(k + NBUF, b)
        return carry

    lax.fori_loop(0, NGRP - 1, _group, 0)

    for k in range(NCHUNK - NBUF, NCHUNK):
        b = k % NBUF
        if k + 1 < NCHUNK:
            bn = (k + 1) % NBUF
            wait_s(bn)
            wait_e(bn)
            gather(bn)
        wait_g(b)
        relu(b)
        wait_d(b)
        pltpu.sync_copy(msgb.at[b], agg_sh.at[dbuf.at[b]], add=True)

    plsc.subcore_barrier()
    pltpu.sync_copy(
        agg_sh.at[pl.ds(sid * ROWS_PT, ROWS_PT)],
        out_hbm.at[pl.ds((cid * NS + sid) * ROWS_PT, ROWS_PT)],
    )


def _segment_msgsum(x2, srcA, dstA, e2):
    mesh = plsc.VectorSubcoreMesh(core_axis_name="c", subcore_axis_name="s")
    fn = pl.kernel(
        _sc_body,
        out_type=jax.ShapeDtypeStruct((NC * N_PAD, DH), jnp.float32),
        mesh=mesh,
        compiler_params=pltpu.CompilerParams(use_tc_tiling_on_sc=False),
        scratch_types=[
            pltpu.VMEM((NBUF, C), jnp.int32),
            pltpu.VMEM((NBUF, C), jnp.int32),
            pltpu.VMEM((NBUF, C, DH), jnp.float32),
            pltpu.VMEM_SHARED((N_PAD, DH), jnp.float32),
            pltpu.SemaphoreType.DMA((NBUF,)),
            pltpu.SemaphoreType.DMA((NBUF,)),
            pltpu.SemaphoreType.DMA((NBUF,)),
            pltpu.SemaphoreType.DMA((NBUF,)),
        ],
    )
    return fn(x2, srcA, dstA, e2)


def _update_body(x_ref, agg_ref, w_ref, b_ref, g_ref, be_ref, o_ref):
    agg = jnp.concatenate(
        [agg_ref[:N_NODES, :], agg_ref[N_PAD:N_PAD + N_NODES, :]], axis=1)
    h = x_ref[...] + agg
    h = jnp.dot(h, w_ref[...], preferred_element_type=jnp.float32) + b_ref[...]
    mean = jnp.mean(h, axis=0, keepdims=True)
    dlt = h - mean
    var = jnp.mean(dlt * dlt, axis=0, keepdims=True)
    h = dlt * lax.rsqrt(var + BN_EPS) * g_ref[...] + be_ref[...]
    o_ref[...] = jnp.maximum(h, 0.0)


def _node_update(x, agg, W_mlp, b_mlp, gamma, beta):
    return pl.pallas_call(
        _update_body,
        out_shape=jax.ShapeDtypeStruct((N_NODES, D), jnp.float32),
    )(x, agg, W_mlp, b_mlp.reshape(1, D), gamma.reshape(1, D),
      beta.reshape(1, D))


def kernel(x, edge_index, edge_attr, W_e, b_e, W_mlp, b_mlp, gamma, beta):
    src = edge_index[0].astype(jnp.int32)
    dst = edge_index[1].astype(jnp.int32)
    x2 = jnp.concatenate([x[:, :DH], x[:, DH:]], axis=0)
    srcg = src.reshape(NS, NCHUNK, C)
    srcA = jnp.concatenate([srcg, srcg + N_NODES], axis=0)
    srcA = srcA.reshape(NC * NS, NCHUNK, C)
    dstg = dst.reshape(NS, NCHUNK, C)
    dstA = jnp.concatenate([dstg, dstg], axis=0).reshape(NC * NS, NCHUNK, C)

    e = _edge_proj(edge_attr, W_e, b_e)
    agg = _segment_msgsum(x2, srcA, dstA, e.reshape(NC * N_EDGES, DH))
    return _node_update(x, agg, W_mlp, b_mlp, gamma, beta)


# speedup vs baseline: 1.9745x; 1.9745x over previous
import jax
import jax.numpy as jnp
from jax import lax
from jax.experimental import pallas as pl
from jax.experimental.pallas import tpu as pltpu
from jax.experimental.pallas import tpu_sc as plsc

N_NODES = 10000
N_EDGES = 320000
D = 128
D_EDGE = 16
BN_EPS = 1e-5

NC = 2
NS = 16
NW = NC * NS
EPW = N_EDGES // NW
C = 40
NCHUNK = EPW // C
N_PAD = 10240
ROWS_PT = N_PAD // NS
LANES = 16
NBUF = 5
NGRP = NCHUNK // NBUF


_BLK1 = 3200


def _edge_proj_body(a_ref, w_ref, b_ref, o_ref):
    o_ref[...] = (
        jnp.dot(a_ref[...], w_ref[...], preferred_element_type=jnp.float32)
        + b_ref[...]
    )


def _edge_proj(edge_attr, W_e, b_e):
    return pl.pallas_call(
        _edge_proj_body,
        grid=(N_EDGES // _BLK1,),
        in_specs=[
            pl.BlockSpec((_BLK1, D_EDGE), lambda i: (i, 0)),
            pl.BlockSpec((D_EDGE, D), lambda i: (0, 0)),
            pl.BlockSpec((1, D), lambda i: (0, 0)),
        ],
        out_specs=pl.BlockSpec((_BLK1, D), lambda i: (i, 0)),
        out_shape=jax.ShapeDtypeStruct((N_EDGES, D), jnp.float32),
    )(edge_attr, W_e, b_e.reshape(1, D))


def _sc_body(x_hbm, src_hbm, dst_hbm, e_hbm, out_hbm,
             sbuf, dbuf, msgb, agg_sh, ssem, dsem, esem, gsem):
    cid = lax.axis_index("c")
    sid = lax.axis_index("s")
    wid = cid * NS + sid
    woff = wid * EPW

    def idx_load(k, b):
        pltpu.async_copy(src_hbm.at[wid, k], sbuf.at[b], ssem.at[b])
        pltpu.async_copy(dst_hbm.at[wid, k], dbuf.at[b], dsem.at[b])

    def wait_s(b):
        pltpu.make_async_copy(src_hbm.at[0, 0], sbuf.at[b], ssem.at[b]).wait()

    def wait_d(b):
        pltpu.make_async_copy(dst_hbm.at[0, 0], dbuf.at[b], dsem.at[b]).wait()

    def e_load(k, b):
        base = pl.multiple_of(woff + k * C, 8)
        pltpu.async_copy(e_hbm.at[pl.ds(base, C)], msgb.at[b], esem.at[b])

    def wait_e(b):
        pltpu.make_async_copy(e_hbm.at[pl.ds(0, C)], msgb.at[b],
                              esem.at[b]).wait()

    def gather(b):
        pltpu.async_copy(x_hbm.at[sbuf.at[b]], msgb.at[b], gsem.at[b],
                         add=True)

    def wait_g(b):
        pltpu.make_async_copy(e_hbm.at[pl.ds(0, C)], msgb.at[b],
                              gsem.at[b]).wait()

    def relu(b):
        def _row(r, c2):
            for rr in range(2):
                for j in range(D // LANES):
                    sl = pl.ds(j * LANES, LANES)
                    msgb[b, 2 * r + rr, sl] = jnp.maximum(
                        msgb[b, 2 * r + rr, sl], 0.0)
            return c2
        lax.fori_loop(0, C // 2, _row, 0)

    zero = jnp.zeros((LANES,), jnp.float32)

    def _zrow(r, carry):
        for j in range(D // LANES):
            msgb[0, r, pl.ds(j * LANES, LANES)] = zero
        return carry

    lax.fori_loop(0, C, _zrow, 0)
    for i in range(ROWS_PT // C):
        pltpu.sync_copy(msgb.at[0],
                        agg_sh.at[pl.ds(sid * ROWS_PT + i * C, C)])

    for b in range(NBUF):
        idx_load(b, b)
        e_load(b, b)
    plsc.subcore_barrier()

    wait_s(0)
    wait_e(0)
    gather(0)

    def _group(g, carry):
        for b in range(NBUF):
            k = g * NBUF + b
            bn = (b + 1) % NBUF
            wait_s(bn)
            wait_e(bn)
            gather(bn)
            wait_g(b)
            relu(b)
            wait_d(b)
            pltpu.sync_copy(msgb.at[b], agg_sh.at[dbuf.at[b]], add=True)
            idx_load(k + NBUF, b)
            e_load(k + NBUF, b)
        return carry

    lax.fori_loop(0, NGRP - 1, _group, 0)

    for k in range(NCHUNK - NBUF, NCHUNK):
        b = k % NBUF
        if k + 1 < NCHUNK:
            bn = (k + 1) % NBUF
            wait_s(bn)
            wait_e(bn)
            gather(bn)
        wait_g(b)
        relu(b)
        wait_d(b)
        pltpu.sync_copy(msgb.at[b], agg_sh.at[dbuf.at[b]], add=True)

    plsc.subcore_barrier()
    pltpu.sync_copy(
        agg_sh.at[pl.ds(sid * ROWS_PT, ROWS_PT)],
        out_hbm.at[pl.ds((cid * NS + sid) * ROWS_PT, ROWS_PT)],
    )


def _segment_msgsum(x, srcA, dstA, e):
    mesh = plsc.VectorSubcoreMesh(core_axis_name="c", subcore_axis_name="s")
    fn = pl.kernel(
        _sc_body,
        out_type=jax.ShapeDtypeStruct((NC * N_PAD, D), jnp.float32),
        mesh=mesh,
        scratch_types=[
            pltpu.VMEM((NBUF, C), jnp.int32),
            pltpu.VMEM((NBUF, C), jnp.int32),
            pltpu.VMEM((NBUF, C, D), jnp.float32),
            pltpu.VMEM_SHARED((N_PAD, D), jnp.float32),
            pltpu.SemaphoreType.DMA((NBUF,)),
            pltpu.SemaphoreType.DMA((NBUF,)),
            pltpu.SemaphoreType.DMA((NBUF,)),
            pltpu.SemaphoreType.DMA((NBUF,)),
        ],
    )
    return fn(x, srcA, dstA, e)


def _update_body(x_ref, agg_ref, w_ref, b_ref, g_ref, be_ref, o_ref):
    h = (x_ref[...] + agg_ref[:N_NODES, :]
         + agg_ref[N_PAD:N_PAD + N_NODES, :])
    h = jnp.dot(h, w_ref[...], preferred_element_type=jnp.float32) + b_ref[...]
    mean = jnp.mean(h, axis=0, keepdims=True)
    dlt = h - mean
    var = jnp.mean(dlt * dlt, axis=0, keepdims=True)
    h = dlt * lax.rsqrt(var + BN_EPS) * g_ref[...] + be_ref[...]
    o_ref[...] = jnp.maximum(h, 0.0)


def _node_update(x, agg, W_mlp, b_mlp, gamma, beta):
    return pl.pallas_call(
        _update_body,
        out_shape=jax.ShapeDtypeStruct((N_NODES, D), jnp.float32),
    )(x, agg, W_mlp, b_mlp.reshape(1, D), gamma.reshape(1, D),
      beta.reshape(1, D))


def kernel(x, edge_index, edge_attr, W_e, b_e, W_mlp, b_mlp, gamma, beta):
    src = edge_index[0].astype(jnp.int32)
    dst = edge_index[1].astype(jnp.int32)
    srcA = src.reshape(NW, NCHUNK, C)
    dstA = dst.reshape(NW, NCHUNK, C)
    e = _edge_proj(edge_attr, W_e, b_e)
    agg = _segment_msgsum(x, srcA, dstA, e)
    return _node_update(x, agg, W_mlp, b_mlp, gamma, beta)


# speedup vs baseline: 2.0958x; 1.0614x over previous
import jax
import jax.numpy as jnp
from jax import lax
from jax.experimental import pallas as pl
from jax.experimental.pallas import tpu as pltpu
from jax.experimental.pallas import tpu_sc as plsc

N_NODES = 10000
N_EDGES = 320000
D = 128
D_EDGE = 16
BN_EPS = 1e-5

NC = 2
NS = 16
NW = NC * NS
EPW = N_EDGES // NW
C = 80
NCHUNK = EPW // C
N_PAD = 10240
ROWS_PT = N_PAD // NS
LANES = 16
NBUF = 4
NG2 = (NCHUNK - NBUF - 1) // NBUF
EPI0 = 1 + NG2 * NBUF


_BLK1 = 3200


def _edge_proj_body(a_ref, w_ref, b_ref, o_ref):
    o_ref[...] = (
        jnp.dot(a_ref[...], w_ref[...], preferred_element_type=jnp.float32)
        + b_ref[...]
    )


def _edge_proj(edge_attr, W_e, b_e):
    return pl.pallas_call(
        _edge_proj_body,
        grid=(N_EDGES // _BLK1,),
        in_specs=[
            pl.BlockSpec((_BLK1, D_EDGE), lambda i: (i, 0)),
            pl.BlockSpec((D_EDGE, D), lambda i: (0, 0)),
            pl.BlockSpec((1, D), lambda i: (0, 0)),
        ],
        out_specs=pl.BlockSpec((_BLK1, D), lambda i: (i, 0)),
        out_shape=jax.ShapeDtypeStruct((N_EDGES, D), jnp.float32),
    )(edge_attr, W_e, b_e.reshape(1, D))


def _sc_body(x_hbm, src_hbm, dst_hbm, e_hbm, out_hbm,
             sbuf, dbuf, msgb, agg_sh, ssem, dsem, esem, gsem, csem):
    cid = lax.axis_index("c")
    sid = lax.axis_index("s")
    wid = cid * NS + sid
    woff = wid * EPW

    def idx_load(k, b):
        base = pl.multiple_of(woff + k * C, 8)
        pltpu.async_copy(src_hbm.at[pl.ds(base, C)], sbuf.at[b], ssem.at[b])
        pltpu.async_copy(dst_hbm.at[pl.ds(base, C)], dbuf.at[b], dsem.at[b])

    def wait_s(b):
        pltpu.make_async_copy(src_hbm.at[pl.ds(0, C)], sbuf.at[b],
                              ssem.at[b]).wait()

    def wait_d(b):
        pltpu.make_async_copy(dst_hbm.at[pl.ds(0, C)], dbuf.at[b],
                              dsem.at[b]).wait()

    def e_load(k, b):
        base = pl.multiple_of(woff + k * C, 8)
        pltpu.async_copy(e_hbm.at[pl.ds(base, C)], msgb.at[b], esem.at[b])

    def wait_e(b):
        pltpu.make_async_copy(e_hbm.at[pl.ds(0, C)], msgb.at[b],
                              esem.at[b]).wait()

    def gather(b):
        pltpu.async_copy(x_hbm.at[sbuf.at[b]], msgb.at[b], gsem.at[b],
                         add=True)

    def wait_g(b):
        pltpu.make_async_copy(e_hbm.at[pl.ds(0, C)], msgb.at[b],
                              gsem.at[b]).wait()

    def scatter(b):
        pltpu.async_copy(msgb.at[b], agg_sh.at[dbuf.at[b]], csem.at[b],
                         add=True)

    def wait_c(b):
        pltpu.make_async_copy(e_hbm.at[pl.ds(0, C)], msgb.at[b],
                              csem.at[b]).wait()

    def relu(b):
        def _row(r, c2):
            for rr in range(2):
                for j in range(D // LANES):
                    sl = pl.ds(j * LANES, LANES)
                    msgb[b, 2 * r + rr, sl] = jnp.maximum(
                        msgb[b, 2 * r + rr, sl], 0.0)
            return c2
        lax.fori_loop(0, C // 2, _row, 0)

    zero = jnp.zeros((LANES,), jnp.float32)

    def _zrow(r, carry):
        for j in range(D // LANES):
            msgb[0, r, pl.ds(j * LANES, LANES)] = zero
        return carry

    lax.fori_loop(0, C, _zrow, 0)
    for i in range(ROWS_PT // C):
        pltpu.sync_copy(msgb.at[0],
                        agg_sh.at[pl.ds(sid * ROWS_PT + i * C, C)])

    for b in range(NBUF - 1):
        idx_load(b, b)
        e_load(b, b)
    plsc.subcore_barrier()

    wait_s(0)
    wait_e(0)
    gather(0)

    idx_load(NBUF - 1, NBUF - 1)
    e_load(NBUF - 1, NBUF - 1)
    wait_s(1)
    wait_e(1)
    gather(1)
    wait_g(0)
    relu(0)
    wait_d(0)
    scatter(0)

    def _group(g, carry):
        for j in range(NBUF):
            k = 1 + g * NBUF + j
            b = (1 + j) % NBUF
            bn = (b + 1) % NBUF
            bp = (b - 1) % NBUF
            wait_c(bp)
            idx_load(k - 1 + NBUF, bp)
            e_load(k - 1 + NBUF, bp)
            wait_s(bn)
            wait_e(bn)
            gather(bn)
            wait_g(b)
            relu(b)
            wait_d(b)
            scatter(b)
        return carry

    lax.fori_loop(0, NG2, _group, 0)

    for k in range(EPI0, NCHUNK):
        b = k % NBUF
        bn = (b + 1) % NBUF
        bp = (b - 1) % NBUF
        if k - 1 + NBUF < NCHUNK:
            wait_c(bp)
            idx_load(k - 1 + NBUF, bp)
            e_load(k - 1 + NBUF, bp)
        if k + 1 < NCHUNK:
            wait_s(bn)
            wait_e(bn)
            gather(bn)
        wait_g(b)
        relu(b)
        wait_d(b)
        scatter(b)

    for b in range(NBUF):
        wait_c(b)

    plsc.subcore_barrier()
    pltpu.sync_copy(
        agg_sh.at[pl.ds(sid * ROWS_PT, ROWS_PT)],
        out_hbm.at[pl.ds((cid * NS + sid) * ROWS_PT, ROWS_PT)],
    )


def _segment_msgsum(x, srcA, dstA, e):
    mesh = plsc.VectorSubcoreMesh(core_axis_name="c", subcore_axis_name="s")
    fn = pl.kernel(
        _sc_body,
        out_type=jax.ShapeDtypeStruct((NC * N_PAD, D), jnp.float32),
        mesh=mesh,
        scratch_types=[
            pltpu.VMEM((NBUF, C), jnp.int32),
            pltpu.VMEM((NBUF, C), jnp.int32),
            pltpu.VMEM((NBUF, C, D), jnp.float32),
            pltpu.VMEM_SHARED((N_PAD, D), jnp.float32),
            pltpu.SemaphoreType.DMA((NBUF,)),
            pltpu.SemaphoreType.DMA((NBUF,)),
            pltpu.SemaphoreType.DMA((NBUF,)),
            pltpu.SemaphoreType.DMA((NBUF,)),
            pltpu.SemaphoreType.DMA((NBUF,)),
        ],
    )
    return fn(x, srcA, dstA, e)


def _update_body(x_ref, agg_ref, w_ref, b_ref, g_ref, be_ref, o_ref):
    h = (x_ref[...] + agg_ref[:N_NODES, :]
         + agg_ref[N_PAD:N_PAD + N_NODES, :])
    h = jnp.dot(h, w_ref[...], preferred_element_type=jnp.float32) + b_ref[...]
    mean = jnp.mean(h, axis=0, keepdims=True)
    dlt = h - mean
    var = jnp.mean(dlt * dlt, axis=0, keepdims=True)
    h = dlt * lax.rsqrt(var + BN_EPS) * g_ref[...] + be_ref[...]
    o_ref[...] = jnp.maximum(h, 0.0)


def _node_update(x, agg, W_mlp, b_mlp, gamma, beta):
    return pl.pallas_call(
        _update_body,
        out_shape=jax.ShapeDtypeStruct((N_NODES, D), jnp.float32),
    )(x, agg, W_mlp, b_mlp.reshape(1, D), gamma.reshape(1, D),
      beta.reshape(1, D))


def kernel(x, edge_index, edge_attr, W_e, b_e, W_mlp, b_mlp, gamma, beta):
    src = edge_index[0].astype(jnp.int32)
    dst = edge_index[1].astype(jnp.int32)
    e = _edge_proj(edge_attr, W_e, b_e)
    agg = _segment_msgsum(x, src, dst, e)
    return _node_update(x, agg, W_mlp, b_mlp, gamma, beta)


# speedup vs baseline: 2.7763x; 1.3247x over previous
import jax
import jax.numpy as jnp
from jax import lax
from jax.experimental import pallas as pl
from jax.experimental.pallas import tpu as pltpu
from jax.experimental.pallas import tpu_sc as plsc

N_NODES = 10000
N_EDGES = 320000
D = 128
D_EDGE = 16
BN_EPS = 1e-5

NC = 2
NS = 16
NW = NC * NS
EPW = N_EDGES // NW
C = 80
NCHUNK = EPW // C
N_PAD = 10240
ROWS_PT = N_PAD // NS
LANES = 16
NBUF = 4
NG2 = (NCHUNK - NBUF - 1) // NBUF
EPI0 = 1 + NG2 * NBUF


_BLK1 = 3200


def _edge_proj_body(at_ref, w_ref, b_ref, o_ref):
    o_ref[...] = (
        lax.dot_general(at_ref[...], w_ref[...], (((0,), (0,)), ((), ())),
                        preferred_element_type=jnp.float32)
        + b_ref[...]
    )


def _edge_proj(edge_attr, W_e, b_e):
    return pl.pallas_call(
        _edge_proj_body,
        grid=(N_EDGES // _BLK1,),
        in_specs=[
            pl.BlockSpec((D_EDGE, _BLK1), lambda i: (0, i)),
            pl.BlockSpec((D_EDGE, D), lambda i: (0, 0)),
            pl.BlockSpec((1, D), lambda i: (0, 0)),
        ],
        out_specs=pl.BlockSpec((_BLK1, D), lambda i: (i, 0)),
        out_shape=jax.ShapeDtypeStruct((N_EDGES, D), jnp.float32),
    )(edge_attr.T, W_e, b_e.reshape(1, D))


def _sc_body(x_hbm, src_hbm, dst_hbm, e_hbm, out_hbm,
             sbuf, dbuf, msgb, agg_sh, ssem, dsem, esem, gsem, csem):
    cid = lax.axis_index("c")
    sid = lax.axis_index("s")
    wid = cid * NS + sid
    woff = wid * EPW

    def idx_load(k, b):
        base = pl.multiple_of(woff + k * C, 8)
        pltpu.async_copy(src_hbm.at[pl.ds(base, C)], sbuf.at[b], ssem.at[b])
        pltpu.async_copy(dst_hbm.at[pl.ds(base, C)], dbuf.at[b], dsem.at[b])

    def wait_s(b):
        pltpu.make_async_copy(src_hbm.at[pl.ds(0, C)], sbuf.at[b],
                              ssem.at[b]).wait()

    def wait_d(b):
        pltpu.make_async_copy(dst_hbm.at[pl.ds(0, C)], dbuf.at[b],
                              dsem.at[b]).wait()

    def e_load(k, b):
        base = pl.multiple_of(woff + k * C, 8)
        pltpu.async_copy(e_hbm.at[pl.ds(base, C)], msgb.at[b], esem.at[b])

    def wait_e(b):
        pltpu.make_async_copy(e_hbm.at[pl.ds(0, C)], msgb.at[b],
                              esem.at[b]).wait()

    def gather(b):
        pltpu.async_copy(x_hbm.at[sbuf.at[b]], msgb.at[b], gsem.at[b],
                         add=True)

    def wait_g(b):
        pltpu.make_async_copy(e_hbm.at[pl.ds(0, C)], msgb.at[b],
                              gsem.at[b]).wait()

    def scatter(b):
        pltpu.async_copy(msgb.at[b], agg_sh.at[dbuf.at[b]], csem.at[b],
                         add=True)

    def wait_c(b):
        pltpu.make_async_copy(e_hbm.at[pl.ds(0, C)], msgb.at[b],
                              csem.at[b]).wait()

    def relu(b):
        def _row(r, c2):
            for rr in range(2):
                for j in range(D // LANES):
                    sl = pl.ds(j * LANES, LANES)
                    msgb[b, 2 * r + rr, sl] = jnp.maximum(
                        msgb[b, 2 * r + rr, sl], 0.0)
            return c2
        lax.fori_loop(0, C // 2, _row, 0)

    zero = jnp.zeros((LANES,), jnp.float32)

    def _zrow(r, carry):
        for j in range(D // LANES):
            msgb[0, r, pl.ds(j * LANES, LANES)] = zero
        return carry

    lax.fori_loop(0, C, _zrow, 0)
    for i in range(ROWS_PT // C):
        pltpu.sync_copy(msgb.at[0],
                        agg_sh.at[pl.ds(sid * ROWS_PT + i * C, C)])

    for b in range(NBUF - 1):
        idx_load(b, b)
        e_load(b, b)
    plsc.subcore_barrier()

    wait_s(0)
    wait_e(0)
    gather(0)

    idx_load(NBUF - 1, NBUF - 1)
    e_load(NBUF - 1, NBUF - 1)
    wait_s(1)
    wait_e(1)
    gather(1)
    wait_g(0)
    relu(0)
    wait_d(0)
    scatter(0)

    def _group(g, carry):
        for j in range(NBUF):
            k = 1 + g * NBUF + j
            b = (1 + j) % NBUF
            bn = (b + 1) % NBUF
            bp = (b - 1) % NBUF
            wait_c(bp)
            idx_load(k - 1 + NBUF, bp)
            e_load(k - 1 + NBUF, bp)
            wait_s(bn)
            wait_e(bn)
            gather(bn)
            wait_g(b)
            relu(b)
            wait_d(b)
            scatter(b)
        return carry

    lax.fori_loop(0, NG2, _group, 0)

    for k in range(EPI0, NCHUNK):
        b = k % NBUF
        bn = (b + 1) % NBUF
        bp = (b - 1) % NBUF
        if k - 1 + NBUF < NCHUNK:
            wait_c(bp)
            idx_load(k - 1 + NBUF, bp)
            e_load(k - 1 + NBUF, bp)
        if k + 1 < NCHUNK:
            wait_s(bn)
            wait_e(bn)
            gather(bn)
        wait_g(b)
        relu(b)
        wait_d(b)
        scatter(b)

    for b in range(NBUF):
        wait_c(b)

    plsc.subcore_barrier()
    pltpu.sync_copy(
        agg_sh.at[pl.ds(sid * ROWS_PT, ROWS_PT)],
        out_hbm.at[pl.ds((cid * NS + sid) * ROWS_PT, ROWS_PT)],
    )


def _segment_msgsum(x, srcA, dstA, e):
    mesh = plsc.VectorSubcoreMesh(core_axis_name="c", subcore_axis_name="s")
    fn = pl.kernel(
        _sc_body,
        out_type=jax.ShapeDtypeStruct((NC * N_PAD, D), jnp.float32),
        mesh=mesh,
        scratch_types=[
            pltpu.VMEM((NBUF, C), jnp.int32),
            pltpu.VMEM((NBUF, C), jnp.int32),
            pltpu.VMEM((NBUF, C, D), jnp.float32),
            pltpu.VMEM_SHARED((N_PAD, D), jnp.float32),
            pltpu.SemaphoreType.DMA((NBUF,)),
            pltpu.SemaphoreType.DMA((NBUF,)),
            pltpu.SemaphoreType.DMA((NBUF,)),
            pltpu.SemaphoreType.DMA((NBUF,)),
            pltpu.SemaphoreType.DMA((NBUF,)),
        ],
    )
    return fn(x, srcA, dstA, e)


def _update_body(x_ref, agg_ref, w_ref, b_ref, g_ref, be_ref, o_ref):
    h = (x_ref[...] + agg_ref[:N_NODES, :]
         + agg_ref[N_PAD:N_PAD + N_NODES, :])
    h = jnp.dot(h, w_ref[...], preferred_element_type=jnp.float32) + b_ref[...]
    mean = jnp.mean(h, axis=0, keepdims=True)
    dlt = h - mean
    var = jnp.mean(dlt * dlt, axis=0, keepdims=True)
    h = dlt * lax.rsqrt(var + BN_EPS) * g_ref[...] + be_ref[...]
    o_ref[...] = jnp.maximum(h, 0.0)


def _node_update(x, agg, W_mlp, b_mlp, gamma, beta):
    return pl.pallas_call(
        _update_body,
        out_shape=jax.ShapeDtypeStruct((N_NODES, D), jnp.float32),
    )(x, agg, W_mlp, b_mlp.reshape(1, D), gamma.reshape(1, D),
      beta.reshape(1, D))


def kernel(x, edge_index, edge_attr, W_e, b_e, W_mlp, b_mlp, gamma, beta):
    src = edge_index[0].astype(jnp.int32)
    dst = edge_index[1].astype(jnp.int32)
    e = _edge_proj(edge_attr, W_e, b_e)
    agg = _segment_msgsum(x, src, dst, e)
    return _node_update(x, agg, W_mlp, b_mlp, gamma, beta)


# speedup vs baseline: 2.7836x; 1.0026x over previous
import jax
import jax.numpy as jnp
from jax import lax
from jax.experimental import pallas as pl
from jax.experimental.pallas import tpu as pltpu
from jax.experimental.pallas import tpu_sc as plsc

N_NODES = 10000
N_EDGES = 320000
D = 128
D_EDGE = 16
BN_EPS = 1e-5

NC = 2
NS = 16
NW = NC * NS
NPH = 2
EPH = N_EDGES // NPH
EPW = EPH // NW
C = 40
NCHUNK = EPW // C
N_PAD = 10240
ROWS_PT = N_PAD // NS
LANES = 16
NBUF = 4
NG2 = (NCHUNK - NBUF - 1) // NBUF
EPI0 = 1 + NG2 * NBUF


_BLK1 = 3200


def _edge_proj_body(at_ref, w_ref, b_ref, o_ref):
    o_ref[...] = (
        lax.dot_general(at_ref[...], w_ref[...], (((0,), (0,)), ((), ())),
                        preferred_element_type=jnp.float32)
        + b_ref[...]
    )


def _edge_proj(edge_attr_t, W_e, b_e, ph):
    nblk = EPH // _BLK1
    return pl.pallas_call(
        _edge_proj_body,
        grid=(nblk,),
        in_specs=[
            pl.BlockSpec((D_EDGE, _BLK1), lambda i: (0, ph * nblk + i)),
            pl.BlockSpec((D_EDGE, D), lambda i: (0, 0)),
            pl.BlockSpec((1, D), lambda i: (0, 0)),
        ],
        out_specs=pl.BlockSpec((_BLK1, D), lambda i: (i, 0)),
        out_shape=jax.ShapeDtypeStruct((EPH, D), jnp.float32),
    )(edge_attr_t, W_e, b_e.reshape(1, D))


def _sc_body(ph, x_hbm, src_hbm, dst_hbm, e_hbm, out_hbm,
             sbuf, dbuf, msgb, agg_sh, ssem, dsem, esem, gsem, csem):
    cid = lax.axis_index("c")
    sid = lax.axis_index("s")
    wid = cid * NS + sid
    woff = wid * EPW
    goff = ph * EPH + woff

    def idx_load(k, b):
        base = pl.multiple_of(goff + k * C, 8)
        pltpu.async_copy(src_hbm.at[pl.ds(base, C)], sbuf.at[b], ssem.at[b])
        pltpu.async_copy(dst_hbm.at[pl.ds(base, C)], dbuf.at[b], dsem.at[b])

    def wait_s(b):
        pltpu.make_async_copy(src_hbm.at[pl.ds(0, C)], sbuf.at[b],
                              ssem.at[b]).wait()

    def wait_d(b):
        pltpu.make_async_copy(dst_hbm.at[pl.ds(0, C)], dbuf.at[b],
                              dsem.at[b]).wait()

    def e_load(k, b):
        base = pl.multiple_of(woff + k * C, 8)
        pltpu.async_copy(e_hbm.at[pl.ds(base, C)], msgb.at[b], esem.at[b])

    def wait_e(b):
        pltpu.make_async_copy(e_hbm.at[pl.ds(0, C)], msgb.at[b],
                              esem.at[b]).wait()

    def gather(b):
        pltpu.async_copy(x_hbm.at[sbuf.at[b]], msgb.at[b], gsem.at[b],
                         add=True)

    def wait_g(b):
        pltpu.make_async_copy(e_hbm.at[pl.ds(0, C)], msgb.at[b],
                              gsem.at[b]).wait()

    def scatter(b):
        pltpu.async_copy(msgb.at[b], agg_sh.at[dbuf.at[b]], csem.at[b],
                         add=True)

    def wait_c(b):
        pltpu.make_async_copy(e_hbm.at[pl.ds(0, C)], msgb.at[b],
                              csem.at[b]).wait()

    def relu(b):
        def _row(r, c2):
            for rr in range(2):
                for j in range(D // LANES):
                    sl = pl.ds(j * LANES, LANES)
                    msgb[b, 2 * r + rr, sl] = jnp.maximum(
                        msgb[b, 2 * r + rr, sl], 0.0)
            return c2
        lax.fori_loop(0, C // 2, _row, 0)

    zero = jnp.zeros((LANES,), jnp.float32)

    def _zrow(r, carry):
        for j in range(D // LANES):
            msgb[0, r, pl.ds(j * LANES, LANES)] = zero
        return carry

    lax.fori_loop(0, C, _zrow, 0)
    for i in range(ROWS_PT // C):
        pltpu.sync_copy(msgb.at[0],
                        agg_sh.at[pl.ds(sid * ROWS_PT + i * C, C)])

    for b in range(NBUF - 1):
        idx_load(b, b)
        e_load(b, b)
    plsc.subcore_barrier()

    wait_s(0)
    wait_e(0)
    gather(0)

    idx_load(NBUF - 1, NBUF - 1)
    e_load(NBUF - 1, NBUF - 1)
    wait_s(1)
    wait_e(1)
    gather(1)
    wait_g(0)
    relu(0)
    wait_d(0)
    scatter(0)

    def _group(g, carry):
        for j in range(NBUF):
            k = 1 + g * NBUF + j
            b = (1 + j) % NBUF
            bn = (b + 1) % NBUF
            bp = (b - 1) % NBUF
            wait_c(bp)
            idx_load(k - 1 + NBUF, bp)
            e_load(k - 1 + NBUF, bp)
            wait_s(bn)
            wait_e(bn)
            gather(bn)
            wait_g(b)
            relu(b)
            wait_d(b)
            scatter(b)
        return carry

    lax.fori_loop(0, NG2, _group, 0)

    for k in range(EPI0, NCHUNK):
        b = k % NBUF
        bn = (b + 1) % NBUF
        bp = (b - 1) % NBUF
        if k - 1 + NBUF < NCHUNK:
            wait_c(bp)
            idx_load(k - 1 + NBUF, bp)
            e_load(k - 1 + NBUF, bp)
        if k + 1 < NCHUNK:
            wait_s(bn)
            wait_e(bn)
            gather(bn)
        wait_g(b)
        relu(b)
        wait_d(b)
        scatter(b)

    for b in range(NBUF):
        wait_c(b)

    plsc.subcore_barrier()
    pltpu.sync_copy(
        agg_sh.at[pl.ds(sid * ROWS_PT, ROWS_PT)],
        out_hbm.at[pl.ds((cid * NS + sid) * ROWS_PT, ROWS_PT)],
    )


def _segment_msgsum(x, srcA, dstA, e, ph):
    import functools
    mesh = plsc.VectorSubcoreMesh(core_axis_name="c", subcore_axis_name="s")
    fn = pl.kernel(
        functools.partial(_sc_body, ph),
        out_type=jax.ShapeDtypeStruct((NC * N_PAD, D), jnp.float32),
        mesh=mesh,
        scratch_types=[
            pltpu.VMEM((NBUF, C), jnp.int32),
            pltpu.VMEM((NBUF, C), jnp.int32),
            pltpu.VMEM((NBUF, C, D), jnp.float32),
            pltpu.VMEM_SHARED((N_PAD, D), jnp.float32),
            pltpu.SemaphoreType.DMA((NBUF,)),
            pltpu.SemaphoreType.DMA((NBUF,)),
            pltpu.SemaphoreType.DMA((NBUF,)),
            pltpu.SemaphoreType.DMA((NBUF,)),
            pltpu.SemaphoreType.DMA((NBUF,)),
        ],
    )
    return fn(x, srcA, dstA, e)


def _update_body(x_ref, agg_ref, agg2_ref, w_ref, b_ref, g_ref, be_ref,
                 o_ref):
    h = (x_ref[...] + agg_ref[:N_NODES, :]
         + agg_ref[N_PAD:N_PAD + N_NODES, :]
         + agg2_ref[:N_NODES, :]
         + agg2_ref[N_PAD:N_PAD + N_NODES, :])
    h = jnp.dot(h, w_ref[...], preferred_element_type=jnp.float32) + b_ref[...]
    mean = jnp.mean(h, axis=0, keepdims=True)
    dlt = h - mean
    var = jnp.mean(dlt * dlt, axis=0, keepdims=True)
    h = dlt * lax.rsqrt(var + BN_EPS) * g_ref[...] + be_ref[...]
    o_ref[...] = jnp.maximum(h, 0.0)


def _node_update(x, agg, agg2, W_mlp, b_mlp, gamma, beta):
    return pl.pallas_call(
        _update_body,
        out_shape=jax.ShapeDtypeStruct((N_NODES, D), jnp.float32),
    )(x, agg, agg2, W_mlp, b_mlp.reshape(1, D), gamma.reshape(1, D),
      beta.reshape(1, D))


def kernel(x, edge_index, edge_attr, W_e, b_e, W_mlp, b_mlp, gamma, beta):
    src = edge_index[0].astype(jnp.int32)
    dst = edge_index[1].astype(jnp.int32)
    ea_t = edge_attr.T
    e0 = _edge_proj(ea_t, W_e, b_e, 0)
    agg0 = _segment_msgsum(x, src, dst, e0, 0)
    e1 = _edge_proj(ea_t, W_e, b_e, 1)
    agg1 = _segment_msgsum(x, src, dst, e1, 1)
    return _node_update(x, agg0, agg1, W_mlp, b_mlp, gamma, beta)
